# baseline (speedup 1.0000x reference)
import jax
import jax.numpy as jnp
from jax.experimental import pallas as pl

_NTA = 8192
_D = 256
_E = 8
_BLK = 512


def _fused_masked_kernel(sym_ref, rho_ref, w_ref, b_ref, out_ref):
    sym = sym_ref[...]
    rho = rho_ref[...]
    onehot = (sym == jax.lax.broadcasted_iota(jnp.int32, (_BLK, _E), 1))
    acc = jnp.dot(onehot.astype(jnp.bfloat16), b_ref[...],
                  preferred_element_type=jnp.float32)
    for e in range(_E):
        m = (sym == e)
        xm = jnp.where(m, rho, 0.0).astype(jnp.bfloat16)
        acc += jnp.dot(xm, w_ref[e], preferred_element_type=jnp.float32)
    out_ref[...] = acc


def kernel(rho, symbols, W, b):
    sym2d = symbols.reshape(_NTA, 1)
    w_bf = W.astype(jnp.bfloat16)
    b_bf = b.astype(jnp.bfloat16)
    grid = _NTA // _BLK
    return pl.pallas_call(
        _fused_masked_kernel,
        grid=(grid,),
        in_specs=[
            pl.BlockSpec((_BLK, 1), lambda i: (i, 0)),
            pl.BlockSpec((_BLK, _D), lambda i: (i, 0)),
            pl.BlockSpec((_E, _D, _D), lambda i: (0, 0, 0)),
            pl.BlockSpec((_E, _D), lambda i: (0, 0)),
        ],
        out_specs=pl.BlockSpec((_BLK, _D), lambda i: (i, 0)),
        out_shape=jax.ShapeDtypeStruct((_NTA, _D), jnp.float32),
    )(sym2d, rho, w_bf, b_bf)
